# Initial kernel scaffold; baseline (speedup 1.0000x reference)
import jax
import jax.numpy as jnp
from jax.experimental import pallas as pl


def kernel(node_feat, edge_feat, edge_index, batch_index, num_sampled_nodes_per_hop, num_sampled_edges_per_hop, W_edge_net, b_edge_net, W_root, bias, bn_gamma, bn_beta):
    raise NotImplementedError("write your pallas kernel here")


# speedup vs baseline: 4.3439x; 4.3439x over previous
import functools

import jax
import jax.numpy as jnp
from jax import lax
from jax.experimental import pallas as pl
from jax.experimental.pallas import tpu as pltpu
from jax.experimental.pallas import tpu_sc as plsc

N_NODES = 10000
N_EDGES = 160000
IN_DIM = 32
OUT_DIM = 32
EDGE_FEAT_DIM = 16
MSG_W = 48

NC = 2
NS = 16
NW = NC * NS
E_PER_W = N_EDGES // NW
CHUNK = 1000
N_CHUNKS = E_PER_W // CHUNK

_sc_mesh = functools.partial(
    plsc.VectorSubcoreMesh, core_axis_name="c", subcore_axis_name="s")


@functools.partial(
    pl.kernel,
    mesh=_sc_mesh(),
    out_type=jax.ShapeDtypeStruct((N_EDGES, IN_DIM), jnp.float32),
    scratch_types=[
        pltpu.VMEM((CHUNK,), jnp.int32),
        pltpu.VMEM((CHUNK, IN_DIM), jnp.float32),
        pltpu.SemaphoreType.DMA,
    ],
    compiler_params=pltpu.CompilerParams(use_tc_tiling_on_sc=False),
)
def _gather_rows(nf_hbm, src_hbm, out_hbm, idx_v, rows_v, sem):
    wid = lax.axis_index("s") * NC + lax.axis_index("c")
    base = pl.multiple_of(wid * E_PER_W, 8)
    for i in range(N_CHUNKS):
        off = pl.multiple_of(base + i * CHUNK, 8)
        pltpu.sync_copy(src_hbm.at[pl.ds(off, CHUNK)], idx_v)
        pltpu.async_copy(nf_hbm.at[idx_v], rows_v, sem).wait()
        pltpu.sync_copy(rows_v, out_hbm.at[pl.ds(off, CHUNK)])


@functools.partial(
    pl.kernel,
    mesh=_sc_mesh(),
    out_type=jax.ShapeDtypeStruct((NC, N_NODES, MSG_W), jnp.float32),
    scratch_types=[
        pltpu.VMEM((CHUNK,), jnp.int32),
        pltpu.VMEM((CHUNK, MSG_W), jnp.float32),
        pltpu.VMEM_SHARED((N_NODES, MSG_W), jnp.float32),
    ],
    compiler_params=pltpu.CompilerParams(use_tc_tiling_on_sc=False),
)
def _scatter_add(msgs_hbm, dst_hbm, zeros_hbm, out_hbm, idx_v, rows_v, acc_sh):
    cid = lax.axis_index("c")
    sid = lax.axis_index("s")

    @pl.when(sid == 0)
    def _():
        pltpu.sync_copy(zeros_hbm, acc_sh)

    plsc.subcore_barrier()

    wid = sid * NC + cid
    base = pl.multiple_of(wid * E_PER_W, 8)
    for i in range(N_CHUNKS):
        off = pl.multiple_of(base + i * CHUNK, 8)
        pltpu.sync_copy(dst_hbm.at[pl.ds(off, CHUNK)], idx_v)
        pltpu.sync_copy(msgs_hbm.at[pl.ds(off, CHUNK)], rows_v)
        pltpu.sync_copy(rows_v, acc_sh.at[idx_v], add=True)

    plsc.subcore_barrier()

    rows_lo = 640
    @pl.when(sid < NS - 1)
    def _():
        r0 = pl.multiple_of(sid * rows_lo, 8)
        pltpu.sync_copy(acc_sh.at[pl.ds(r0, rows_lo)],
                        out_hbm.at[cid, pl.ds(r0, rows_lo)])

    @pl.when(sid == NS - 1)
    def _():
        pltpu.sync_copy(acc_sh.at[pl.ds((NS - 1) * rows_lo, N_NODES - (NS - 1) * rows_lo)],
                        out_hbm.at[cid, pl.ds((NS - 1) * rows_lo, N_NODES - (NS - 1) * rows_lo)])


BE = 2000

def _edge_mm_body(ef_ref, x_ref, w4_ref, bmat_ref, r_ref, s_ref, out_ref):
    x = x_ref[...]
    p = lax.dot_general(x, w4_ref[...], (((1,), (0,)), ((), ())),
                        preferred_element_type=jnp.float32)
    ef_exp = lax.dot_general(ef_ref[...], r_ref[...], (((1,), (0,)), ((), ())),
                             preferred_element_type=jnp.float32)
    q = p * ef_exp
    acc = lax.dot_general(q, s_ref[...], (((1,), (0,)), ((), ())),
                          preferred_element_type=jnp.float32)
    acc = acc + lax.dot_general(x, bmat_ref[...], (((1,), (0,)), ((), ())),
                                preferred_element_type=jnp.float32)
    out_ref[:, :OUT_DIM] = acc
    out_ref[:, OUT_DIM:] = jnp.ones((BE, MSG_W - OUT_DIM), jnp.float32)


def _edge_matmul(edge_feat, x_src, w4, bmat, rmat, smat):
    return pl.pallas_call(
        _edge_mm_body,
        grid=(N_EDGES // BE,),
        in_specs=[
            pl.BlockSpec((BE, EDGE_FEAT_DIM), lambda i: (i, 0)),
            pl.BlockSpec((BE, IN_DIM), lambda i: (i, 0)),
            pl.BlockSpec((IN_DIM, EDGE_FEAT_DIM * OUT_DIM), lambda i: (0, 0)),
            pl.BlockSpec((IN_DIM, OUT_DIM), lambda i: (0, 0)),
            pl.BlockSpec((EDGE_FEAT_DIM, EDGE_FEAT_DIM * OUT_DIM), lambda i: (0, 0)),
            pl.BlockSpec((EDGE_FEAT_DIM * OUT_DIM, OUT_DIM), lambda i: (0, 0)),
        ],
        out_specs=pl.BlockSpec((BE, MSG_W), lambda i: (i, 0)),
        out_shape=jax.ShapeDtypeStruct((N_EDGES, MSG_W), jnp.float32),
    )(edge_feat, x_src, w4, bmat, rmat, smat)


def _finalize_body(p_ref, nf_ref, wr_ref, b_ref, g_ref, bt_ref, out_ref):
    s0 = p_ref[0]
    s1 = p_ref[1]
    summed = s0[:, :OUT_DIM] + s1[:, :OUT_DIM]
    cnt = s0[:, OUT_DIM:OUT_DIM + 1] + s1[:, OUT_DIM:OUT_DIM + 1]
    aggr = summed / jnp.maximum(cnt, 1.0)
    out = aggr + lax.dot_general(nf_ref[...], wr_ref[...],
                                 (((1,), (0,)), ((), ())),
                                 preferred_element_type=jnp.float32) + b_ref[...]
    mean = jnp.mean(out, axis=0, keepdims=True)
    var = jnp.mean((out - mean) ** 2, axis=0, keepdims=True)
    out = (out - mean) * lax.rsqrt(var + 1e-5) * g_ref[...] + bt_ref[...]
    out_ref[...] = jnp.where(out >= 0, out, 0.01 * out)


def _finalize(partials, node_feat, w_root, bias, gamma, beta):
    return pl.pallas_call(
        _finalize_body,
        out_shape=jax.ShapeDtypeStruct((N_NODES, OUT_DIM), jnp.float32),
    )(partials, node_feat, w_root,
      bias.reshape(1, OUT_DIM), gamma.reshape(1, OUT_DIM), beta.reshape(1, OUT_DIM))


def kernel(node_feat, edge_feat, edge_index, batch_index,
           num_sampled_nodes_per_hop, num_sampled_edges_per_hop,
           W_edge_net, b_edge_net, W_root, bias, bn_gamma, bn_beta):
    src = edge_index[0].astype(jnp.int32)
    dst = edge_index[1].astype(jnp.int32)
    w4 = W_edge_net.reshape(EDGE_FEAT_DIM, IN_DIM, OUT_DIM).transpose(1, 0, 2) \
                   .reshape(IN_DIM, EDGE_FEAT_DIM * OUT_DIM)
    bmat = b_edge_net.reshape(IN_DIM, OUT_DIM)
    zeros = jnp.zeros((N_NODES, MSG_W), jnp.float32)
    f_ids = jnp.arange(EDGE_FEAT_DIM * OUT_DIM, dtype=jnp.int32) // OUT_DIM
    rmat = (f_ids[None, :] == jnp.arange(EDGE_FEAT_DIM, dtype=jnp.int32)[:, None]
            ).astype(jnp.float32)
    o_ids = jnp.arange(EDGE_FEAT_DIM * OUT_DIM, dtype=jnp.int32) % OUT_DIM
    smat = (o_ids[:, None] == jnp.arange(OUT_DIM, dtype=jnp.int32)[None, :]
            ).astype(jnp.float32)

    x_src = _gather_rows(node_feat, src)
    msgs48 = _edge_matmul(edge_feat, x_src, w4, bmat, rmat, smat)
    partials = _scatter_add(msgs48, dst, zeros)
    out = _finalize(partials, node_feat, W_root, bias, bn_gamma, bn_beta)
    return (out, edge_index, edge_feat)


# speedup vs baseline: 5.3584x; 1.2336x over previous
import functools

import jax
import jax.numpy as jnp
from jax import lax
from jax.experimental import pallas as pl
from jax.experimental.pallas import tpu as pltpu
from jax.experimental.pallas import tpu_sc as plsc

N_NODES = 10000
N_EDGES = 160000
IN_DIM = 32
OUT_DIM = 32
EDGE_FEAT_DIM = 16
PK = 4
FD = EDGE_FEAT_DIM * OUT_DIM

NC = 2
NS = 16
NW = NC * NS
E_PER_W = N_EDGES // NW
CHUNK = 1000
N_CHUNKS = E_PER_W // CHUNK

_sc_mesh = functools.partial(
    plsc.VectorSubcoreMesh, core_axis_name="c", subcore_axis_name="s")
_sc_params = pltpu.CompilerParams(use_tc_tiling_on_sc=False)


@functools.partial(
    pl.kernel,
    mesh=_sc_mesh(),
    out_type=jax.ShapeDtypeStruct((N_EDGES, IN_DIM), jnp.float32),
    scratch_types=[
        pltpu.VMEM((CHUNK,), jnp.int32),
        pltpu.VMEM((CHUNK, IN_DIM), jnp.float32),
        pltpu.SemaphoreType.DMA,
    ],
    compiler_params=_sc_params,
)
def _gather_rows(nf_hbm, src_hbm, out_hbm, idx_v, rows_v, sem):
    wid = lax.axis_index("s") * NC + lax.axis_index("c")
    base = pl.multiple_of(wid * E_PER_W, 8)
    for i in range(N_CHUNKS):
        off = pl.multiple_of(base + i * CHUNK, 8)
        pltpu.sync_copy(src_hbm.at[pl.ds(off, CHUNK)], idx_v)
        pltpu.async_copy(nf_hbm.at[idx_v], rows_v, sem).wait()
        pltpu.sync_copy(rows_v, out_hbm.at[pl.ds(off, CHUNK)])


@functools.partial(
    pl.kernel,
    mesh=_sc_mesh(),
    out_type=[jax.ShapeDtypeStruct((NC, N_NODES, OUT_DIM), jnp.float32),
              jax.ShapeDtypeStruct((NC, N_NODES, OUT_DIM), jnp.float32)],
    scratch_types=[
        pltpu.VMEM((CHUNK,), jnp.int32),
        pltpu.VMEM((CHUNK, OUT_DIM), jnp.float32),
        pltpu.VMEM((CHUNK, OUT_DIM), jnp.float32),
        pltpu.VMEM_SHARED((N_NODES, OUT_DIM), jnp.float32),
        pltpu.VMEM_SHARED((N_NODES, OUT_DIM), jnp.float32),
    ],
    compiler_params=_sc_params,
)
def _scatter_add(msgs_hbm, dst_hbm, zeros_hbm, ones_hbm,
                 sum_hbm, cnt_hbm, idx_v, rows_v, ones_v, acc_sh, cnt_sh):
    cid = lax.axis_index("c")
    sid = lax.axis_index("s")

    pltpu.sync_copy(ones_hbm, ones_v)

    @pl.when(sid == 0)
    def _():
        pltpu.sync_copy(zeros_hbm, acc_sh)

    @pl.when(sid == 1)
    def _():
        pltpu.sync_copy(zeros_hbm, cnt_sh)

    plsc.subcore_barrier()

    wid = sid * NC + cid
    base = pl.multiple_of(wid * E_PER_W, 8)
    for i in range(N_CHUNKS):
        off = pl.multiple_of(base + i * CHUNK, 8)
        pltpu.sync_copy(dst_hbm.at[pl.ds(off, CHUNK)], idx_v)
        pltpu.sync_copy(msgs_hbm.at[pl.ds(off, CHUNK)], rows_v)
        pltpu.sync_copy(rows_v, acc_sh.at[idx_v], add=True)
        pltpu.sync_copy(ones_v, cnt_sh.at[idx_v], add=True)

    plsc.subcore_barrier()

    rows_lo = 640
    r0 = pl.multiple_of(sid * rows_lo, 8)
    last = N_NODES - (NS - 1) * rows_lo

    @pl.when(sid < NS - 1)
    def _():
        pltpu.sync_copy(acc_sh.at[pl.ds(r0, rows_lo)],
                        sum_hbm.at[cid, pl.ds(r0, rows_lo)])
        pltpu.sync_copy(cnt_sh.at[pl.ds(r0, rows_lo)],
                        cnt_hbm.at[cid, pl.ds(r0, rows_lo)])

    @pl.when(sid == NS - 1)
    def _():
        pltpu.sync_copy(acc_sh.at[pl.ds((NS - 1) * rows_lo, last)],
                        sum_hbm.at[cid, pl.ds((NS - 1) * rows_lo, last)])
        pltpu.sync_copy(cnt_sh.at[pl.ds((NS - 1) * rows_lo, last)],
                        cnt_hbm.at[cid, pl.ds((NS - 1) * rows_lo, last)])


BE = 3200
B4 = BE // PK

def _edge_mm_body(ef_ref, x_ref, w_ref, b_ref, r_ref, s_ref, out_ref):
    x = x_ref[...]
    p = lax.dot_general(x, w_ref[...], (((1,), (0,)), ((), ())),
                        preferred_element_type=jnp.float32)
    ef_exp = lax.dot_general(ef_ref[...], r_ref[...], (((1,), (0,)), ((), ())),
                             preferred_element_type=jnp.float32)
    q = p * ef_exp
    acc = lax.dot_general(q, s_ref[...], (((1,), (0,)), ((), ())),
                          preferred_element_type=jnp.float32)
    acc = acc + lax.dot_general(x, b_ref[...], (((1,), (0,)), ((), ())),
                                preferred_element_type=jnp.float32)
    out_ref[...] = acc


def _edge_matmul(ef4, x4, w4blk, bblk, r4, s4):
    return pl.pallas_call(
        _edge_mm_body,
        grid=(N_EDGES // BE,),
        in_specs=[
            pl.BlockSpec((B4, PK * EDGE_FEAT_DIM), lambda i: (i, 0)),
            pl.BlockSpec((B4, PK * IN_DIM), lambda i: (i, 0)),
            pl.BlockSpec((PK * IN_DIM, PK * FD), lambda i: (0, 0)),
            pl.BlockSpec((PK * IN_DIM, PK * OUT_DIM), lambda i: (0, 0)),
            pl.BlockSpec((PK * EDGE_FEAT_DIM, PK * FD), lambda i: (0, 0)),
            pl.BlockSpec((PK * FD, PK * OUT_DIM), lambda i: (0, 0)),
        ],
        out_specs=pl.BlockSpec((B4, PK * OUT_DIM), lambda i: (i, 0)),
        out_shape=jax.ShapeDtypeStruct((N_EDGES // PK, PK * OUT_DIM), jnp.float32),
    )(ef4, x4, w4blk, bblk, r4, s4)


def _finalize_body(s_ref, c_ref, nf_ref, wr_ref, b_ref, g_ref, bt_ref, out_ref):
    summed = s_ref[0] + s_ref[1]
    cnt = c_ref[0] + c_ref[1]
    aggr = summed / jnp.maximum(cnt, 1.0)
    out = aggr + lax.dot_general(nf_ref[...], wr_ref[...],
                                 (((1,), (0,)), ((), ())),
                                 preferred_element_type=jnp.float32) + b_ref[...]
    mean = jnp.mean(out, axis=0, keepdims=True)
    var = jnp.mean((out - mean) ** 2, axis=0, keepdims=True)
    out = (out - mean) * lax.rsqrt(var + 1e-5) * g_ref[...] + bt_ref[...]
    out_ref[...] = jnp.where(out >= 0, out, 0.01 * out)


def _finalize(sums, cnts, node_feat, w_root, bias, gamma, beta):
    return pl.pallas_call(
        _finalize_body,
        out_shape=jax.ShapeDtypeStruct((N_NODES, OUT_DIM), jnp.float32),
    )(sums, cnts, node_feat, w_root,
      bias.reshape(1, OUT_DIM), gamma.reshape(1, OUT_DIM), beta.reshape(1, OUT_DIM))


def kernel(node_feat, edge_feat, edge_index, batch_index,
           num_sampled_nodes_per_hop, num_sampled_edges_per_hop,
           W_edge_net, b_edge_net, W_root, bias, bn_gamma, bn_beta):
    src = edge_index[0].astype(jnp.int32)
    dst = edge_index[1].astype(jnp.int32)
    w4 = W_edge_net.reshape(EDGE_FEAT_DIM, IN_DIM, OUT_DIM).transpose(1, 0, 2) \
                   .reshape(IN_DIM, FD)
    bmat = b_edge_net.reshape(IN_DIM, OUT_DIM)
    f_ids = jnp.arange(FD, dtype=jnp.int32) // OUT_DIM
    rmat = (f_ids[None, :] == jnp.arange(EDGE_FEAT_DIM, dtype=jnp.int32)[:, None]
            ).astype(jnp.float32)
    o_ids = jnp.arange(FD, dtype=jnp.int32) % OUT_DIM
    smat = (o_ids[:, None] == jnp.arange(OUT_DIM, dtype=jnp.int32)[None, :]
            ).astype(jnp.float32)
    eye4 = jnp.eye(PK, dtype=jnp.float32)
    w4blk = jnp.kron(eye4, w4)
    bblk = jnp.kron(eye4, bmat)
    r4 = jnp.kron(eye4, rmat)
    s4 = jnp.kron(eye4, smat)

    zeros = jnp.zeros((N_NODES, OUT_DIM), jnp.float32)
    ones = jnp.ones((CHUNK, OUT_DIM), jnp.float32)

    x_src = _gather_rows(node_feat, src)
    x4 = x_src.reshape(N_EDGES // PK, PK * IN_DIM)
    ef4 = edge_feat.reshape(N_EDGES // PK, PK * EDGE_FEAT_DIM)
    msgs4 = _edge_matmul(ef4, x4, w4blk, bblk, r4, s4)
    msgs = msgs4.reshape(N_EDGES, OUT_DIM)
    sums, cnts = _scatter_add(msgs, dst, zeros, ones)
    out = _finalize(sums, cnts, node_feat, W_root, bias, bn_gamma, bn_beta)
    return (out, edge_index, edge_feat)


# speedup vs baseline: 5.5550x; 1.0367x over previous
import functools

import jax
import jax.numpy as jnp
from jax import lax
from jax.experimental import pallas as pl
from jax.experimental.pallas import tpu as pltpu
from jax.experimental.pallas import tpu_sc as plsc

N_NODES = 10000
N_EDGES = 160000
IN_DIM = 32
OUT_DIM = 32
EDGE_FEAT_DIM = 16
PK = 4
FD = EDGE_FEAT_DIM * OUT_DIM

NC = 2
NS = 16
NW = NC * NS
E_PER_W = N_EDGES // NW
CHUNK = 1000
N_CHUNKS = E_PER_W // CHUNK

_sc_mesh = functools.partial(
    plsc.VectorSubcoreMesh, core_axis_name="c", subcore_axis_name="s")
_sc_params = pltpu.CompilerParams(use_tc_tiling_on_sc=False)


@functools.partial(
    pl.kernel,
    mesh=_sc_mesh(),
    out_type=jax.ShapeDtypeStruct((N_EDGES, IN_DIM), jnp.float32),
    scratch_types=[
        pltpu.VMEM((CHUNK,), jnp.int32),
        pltpu.VMEM((CHUNK, IN_DIM), jnp.float32),
        pltpu.SemaphoreType.DMA,
    ],
    compiler_params=_sc_params,
)
def _gather_rows(nf_hbm, src_hbm, out_hbm, idx_v, rows_v, sem):
    wid = lax.axis_index("s") * NC + lax.axis_index("c")
    base = pl.multiple_of(wid * E_PER_W, 8)
    for i in range(N_CHUNKS):
        off = pl.multiple_of(base + i * CHUNK, 8)
        pltpu.sync_copy(src_hbm.at[pl.ds(off, CHUNK)], idx_v)
        pltpu.async_copy(nf_hbm.at[idx_v], rows_v, sem).wait()
        pltpu.sync_copy(rows_v, out_hbm.at[pl.ds(off, CHUNK)])


@functools.partial(
    pl.kernel,
    mesh=_sc_mesh(),
    out_type=[jax.ShapeDtypeStruct((NC, N_NODES, OUT_DIM), jnp.float32),
              jax.ShapeDtypeStruct((NC, N_NODES, OUT_DIM), jnp.float32)],
    scratch_types=[
        pltpu.VMEM((CHUNK,), jnp.int32),
        pltpu.VMEM((CHUNK, OUT_DIM), jnp.float32),
        pltpu.VMEM((CHUNK, OUT_DIM), jnp.float32),
        pltpu.VMEM_SHARED((N_NODES, OUT_DIM), jnp.float32),
        pltpu.VMEM_SHARED((N_NODES, OUT_DIM), jnp.float32),
    ],
    compiler_params=_sc_params,
)
def _scatter_add(msgs_hbm, dst_hbm, zeros_hbm, ones_hbm,
                 sum_hbm, cnt_hbm, idx_v, rows_v, ones_v, acc_sh, cnt_sh):
    sum_flat = sum_hbm
    cnt_flat = cnt_hbm
    cid = lax.axis_index("c")
    sid = lax.axis_index("s")

    pltpu.sync_copy(ones_hbm, ones_v)

    @pl.when(sid == 0)
    def _():
        pltpu.sync_copy(zeros_hbm, acc_sh)

    @pl.when(sid == 1)
    def _():
        pltpu.sync_copy(zeros_hbm, cnt_sh)

    plsc.subcore_barrier()

    wid = sid * NC + cid
    base = pl.multiple_of(wid * E_PER_W, 8)
    for i in range(N_CHUNKS):
        off = pl.multiple_of(base + i * CHUNK, 8)
        pltpu.sync_copy(dst_hbm.at[pl.ds(off, CHUNK)], idx_v)
        pltpu.sync_copy(msgs_hbm.at[pl.ds(off, CHUNK)], rows_v)
        pltpu.sync_copy(rows_v, acc_sh.at[idx_v], add=True)
        pltpu.sync_copy(ones_v, cnt_sh.at[idx_v], add=True)

    plsc.subcore_barrier()

    rows_lo = 640
    r0 = pl.multiple_of(sid * rows_lo, 8)
    last = N_NODES - (NS - 1) * rows_lo

    @pl.when(sid < NS - 1)
    def _():
        pltpu.sync_copy(acc_sh.at[pl.ds(r0, rows_lo)],
                        sum_flat.at[cid, pl.ds(r0, rows_lo)])
        pltpu.sync_copy(cnt_sh.at[pl.ds(r0, rows_lo)],
                        cnt_flat.at[cid, pl.ds(r0, rows_lo)])

    @pl.when(sid == NS - 1)
    def _():
        pltpu.sync_copy(acc_sh.at[pl.ds((NS - 1) * rows_lo, last)],
                        sum_flat.at[cid, pl.ds((NS - 1) * rows_lo, last)])
        pltpu.sync_copy(cnt_sh.at[pl.ds((NS - 1) * rows_lo, last)],
                        cnt_flat.at[cid, pl.ds((NS - 1) * rows_lo, last)])


BE = 3200
B4 = BE // PK

def _edge_mm_body(ef_ref, x_ref, w_ref, b_ref, r_ref, s_ref, out_ref):
    x = x_ref[...]
    p = lax.dot_general(x, w_ref[...], (((1,), (0,)), ((), ())),
                        preferred_element_type=jnp.float32)
    ef_exp = lax.dot_general(ef_ref[...], r_ref[...], (((1,), (0,)), ((), ())),
                             preferred_element_type=jnp.float32)
    q = p * ef_exp
    acc = lax.dot_general(q, s_ref[...], (((1,), (0,)), ((), ())),
                          preferred_element_type=jnp.float32)
    acc = acc + lax.dot_general(x, b_ref[...], (((1,), (0,)), ((), ())),
                                preferred_element_type=jnp.float32)
    out_ref[...] = acc


def _edge_matmul(ef4, x4, w4blk, bblk, r4, s4):
    return pl.pallas_call(
        _edge_mm_body,
        grid=(N_EDGES // BE,),
        in_specs=[
            pl.BlockSpec((B4, PK * EDGE_FEAT_DIM), lambda i: (i, 0)),
            pl.BlockSpec((B4, PK * IN_DIM), lambda i: (i, 0)),
            pl.BlockSpec((PK * IN_DIM, PK * FD), lambda i: (0, 0)),
            pl.BlockSpec((PK * IN_DIM, PK * OUT_DIM), lambda i: (0, 0)),
            pl.BlockSpec((PK * EDGE_FEAT_DIM, PK * FD), lambda i: (0, 0)),
            pl.BlockSpec((PK * FD, PK * OUT_DIM), lambda i: (0, 0)),
        ],
        out_specs=pl.BlockSpec((B4, PK * OUT_DIM), lambda i: (i, 0)),
        out_shape=jax.ShapeDtypeStruct((N_EDGES // PK, PK * OUT_DIM), jnp.float32),
    )(ef4, x4, w4blk, bblk, r4, s4)


def _lane_fold(v):
    return (v[:, 0 * OUT_DIM:1 * OUT_DIM] + v[:, 1 * OUT_DIM:2 * OUT_DIM]
            + v[:, 2 * OUT_DIM:3 * OUT_DIM] + v[:, 3 * OUT_DIM:4 * OUT_DIM])


def _finalize_body(s_ref, c_ref, nf_ref, wr_ref, b_ref, g_ref, bt_ref, out_ref):
    summed = s_ref[0] + s_ref[1]
    cnt = c_ref[0] + c_ref[1]
    aggr = summed / jnp.maximum(cnt, 1.0)
    out = aggr + lax.dot_general(nf_ref[...], wr_ref[...],
                                 (((1,), (0,)), ((), ())),
                                 preferred_element_type=jnp.float32) + b_ref[...]
    m32 = _lane_fold(jnp.sum(out, axis=0, keepdims=True)) / N_NODES
    mean = jnp.concatenate([m32] * PK, axis=1)
    d = out - mean
    v32 = _lane_fold(jnp.sum(d * d, axis=0, keepdims=True)) / N_NODES
    var = jnp.concatenate([v32] * PK, axis=1)
    out = d * lax.rsqrt(var + 1e-5) * g_ref[...] + bt_ref[...]
    out_ref[...] = jnp.where(out >= 0, out, 0.01 * out)


def _finalize(sums4, cnts4, nf4, wrblk, bias4, gamma4, beta4):
    return pl.pallas_call(
        _finalize_body,
        out_shape=jax.ShapeDtypeStruct((N_NODES // PK, PK * OUT_DIM), jnp.float32),
    )(sums4, cnts4, nf4, wrblk, bias4, gamma4, beta4)


def kernel(node_feat, edge_feat, edge_index, batch_index,
           num_sampled_nodes_per_hop, num_sampled_edges_per_hop,
           W_edge_net, b_edge_net, W_root, bias, bn_gamma, bn_beta):
    src = edge_index[0].astype(jnp.int32)
    dst = edge_index[1].astype(jnp.int32)
    w4 = W_edge_net.reshape(EDGE_FEAT_DIM, IN_DIM, OUT_DIM).transpose(1, 0, 2) \
                   .reshape(IN_DIM, FD)
    bmat = b_edge_net.reshape(IN_DIM, OUT_DIM)
    f_ids = jnp.arange(FD, dtype=jnp.int32) // OUT_DIM
    rmat = (f_ids[None, :] == jnp.arange(EDGE_FEAT_DIM, dtype=jnp.int32)[:, None]
            ).astype(jnp.float32)
    o_ids = jnp.arange(FD, dtype=jnp.int32) % OUT_DIM
    smat = (o_ids[:, None] == jnp.arange(OUT_DIM, dtype=jnp.int32)[None, :]
            ).astype(jnp.float32)
    eye4 = jnp.eye(PK, dtype=jnp.float32)
    w4blk = jnp.kron(eye4, w4)
    bblk = jnp.kron(eye4, bmat)
    r4 = jnp.kron(eye4, rmat)
    s4 = jnp.kron(eye4, smat)

    zeros = jnp.zeros((N_NODES, OUT_DIM), jnp.float32)
    ones = jnp.ones((CHUNK, OUT_DIM), jnp.float32)

    x_src = _gather_rows(node_feat, src)
    x4 = x_src.reshape(-1).reshape(N_EDGES // PK, PK * IN_DIM)
    ef4 = edge_feat.reshape(N_EDGES // PK, PK * EDGE_FEAT_DIM)
    msgs4 = _edge_matmul(ef4, x4, w4blk, bblk, r4, s4)
    msgs = msgs4.reshape(N_EDGES, OUT_DIM)
    sums, cnts = _scatter_add(msgs, dst, zeros, ones)
    sums4 = sums.reshape(-1).reshape(NC, N_NODES // PK, PK * OUT_DIM)
    cnts4 = cnts.reshape(-1).reshape(NC, N_NODES // PK, PK * OUT_DIM)
    nf4 = node_feat.reshape(N_NODES // PK, PK * IN_DIM)
    wrblk = jnp.kron(eye4, W_root)
    bias4 = jnp.tile(bias.reshape(1, OUT_DIM), (1, PK))
    gamma4 = jnp.tile(bn_gamma.reshape(1, OUT_DIM), (1, PK))
    beta4 = jnp.tile(bn_beta.reshape(1, OUT_DIM), (1, PK))
    out4 = _finalize(sums4, cnts4, nf4, wrblk, bias4, gamma4, beta4)
    out = out4.reshape(N_NODES, OUT_DIM)
    return (out, edge_index, edge_feat)
